# Initial kernel scaffold; baseline (speedup 1.0000x reference)
import jax
import jax.numpy as jnp
from jax.experimental import pallas as pl


def kernel(x, w1, b1, g1, bt1, w2, b2, g2, bt2):
    raise NotImplementedError("write your pallas kernel here")


# speedup vs baseline: 1.3820x; 1.3820x over previous
import functools

import jax
import jax.numpy as jnp
from jax import lax
from jax.experimental import pallas as pl
from jax.experimental.pallas import tpu as pltpu

_VMEM_LIMIT = 64 * 1024 * 1024


def _round_up(x, m):
    return (x + m - 1) // m * m


def _pool_kernel(x_ref, o_ref):
    t = x_ref[0, :, 0]
    C, _, H, W = t.shape
    a = jnp.maximum(t[:, 0], t[:, 1])
    b = jnp.transpose(a, (1, 2, 0))
    b = b.reshape(H // 2, 2, W, C)
    b = jnp.maximum(b[:, 0], b[:, 1])
    b = b.reshape(H // 2, W // 2, 2, C)
    b = jnp.maximum(b[:, :, 0], b[:, :, 1])
    o_ref[0, 0] = b.astype(o_ref.dtype)


def _pool_ncdhw_to_ndhwc(x):
    N, C, D, H, W = x.shape
    D2, H2, W2 = D // 2, H // 2, W // 2
    x6 = x.reshape(N, C, D2, 2, H, W)
    return pl.pallas_call(
        _pool_kernel,
        grid=(N, D2),
        in_specs=[pl.BlockSpec((1, C, 1, 2, H, W),
                               lambda n, d: (n, 0, d, 0, 0, 0))],
        out_specs=pl.BlockSpec((1, 1, H2, W2, C), lambda n, d: (n, d, 0, 0, 0)),
        out_shape=jax.ShapeDtypeStruct((N, D2, H2, W2, C), jnp.bfloat16),
        compiler_params=pltpu.CompilerParams(
            dimension_semantics=("parallel", "parallel"),
            vmem_limit_bytes=_VMEM_LIMIT),
    )(x6)


def _conv_kernel(xm_ref, x0_ref, xp_ref, sc_ref, sh_ref, w_ref, b_ref,
                 y_ref, sum_ref, sq_ref, p_ref, s_ref, *, apply_norm):
    d = pl.program_id(1)
    n_d = pl.num_programs(1)
    H, W, Cout = y_ref.shape[2], y_ref.shape[3], y_ref.shape[4]
    Cin = xm_ref.shape[-1]
    Wp = p_ref.shape[2]

    @pl.when(d == 0)
    def _():
        p_ref[...] = jnp.zeros_like(p_ref)
        sum_ref[...] = jnp.zeros_like(sum_ref)
        sq_ref[...] = jnp.zeros_like(sq_ref)

    for kd, slab_ref in enumerate((xm_ref, x0_ref, xp_ref)):
        s = slab_ref[0, 0]
        if apply_norm:
            s = jnp.maximum(s.astype(jnp.float32) * sc_ref[...]
                            + sh_ref[...], 0.0)
        dd = d + (kd - 1)
        valid = jnp.logical_and(dd >= 0, dd < n_d).astype(jnp.float32)
        p_ref[kd, 1:H + 1, 1:W + 1, :] = (s * valid).astype(p_ref.dtype)

    for kd in range(3):
        for kh in range(3):
            j = kd * 3 + kh
            s_ref[:, :, j * Cin:(j + 1) * Cin] = p_ref[kd, kh:kh + H, :, :]

    S = s_ref[...].reshape(H * Wp, 9 * Cin)
    R = jnp.dot(S, w_ref[...], preferred_element_type=jnp.float32)
    R = R.reshape(H, Wp, 3 * Cout)
    acc = (R[:, 0:W, 0:Cout]
           + R[:, 1:W + 1, Cout:2 * Cout]
           + R[:, 2:W + 2, 2 * Cout:3 * Cout])
    y = acc + b_ref[...]
    y_ref[0, 0] = y.astype(y_ref.dtype)

    sum_ref[...] += jnp.sum(y, axis=(0, 1)).reshape(1, 1, Cout)
    sq_ref[...] += jnp.sum(y * y, axis=(0, 1)).reshape(1, 1, Cout)


def _conv3d_bn_stats(x5, wn, b, cout, in_scale=None, in_shift=None):
    N, Dc, H, W, Cin = x5.shape
    Wp = _round_up(W + 2, 8)

    apply_norm = in_scale is not None
    if not apply_norm:
        in_scale = jnp.ones((1, Cin), jnp.float32)
        in_shift = jnp.zeros((1, Cin), jnp.float32)
    bp = b.astype(jnp.float32).reshape(1, cout)

    def slab_spec(off):
        return pl.BlockSpec(
            (1, 1, H, W, Cin),
            lambda n, d: (n, jnp.clip(d + off, 0, Dc - 1), 0, 0, 0))

    y, s1, s2 = pl.pallas_call(
        functools.partial(_conv_kernel, apply_norm=apply_norm),
        grid=(N, Dc),
        in_specs=[
            slab_spec(-1), slab_spec(0), slab_spec(1),
            pl.BlockSpec((1, Cin), lambda n, d: (0, 0)),
            pl.BlockSpec((1, Cin), lambda n, d: (0, 0)),
            pl.BlockSpec((9 * Cin, 3 * cout), lambda n, d: (0, 0)),
            pl.BlockSpec((1, cout), lambda n, d: (0, 0)),
        ],
        out_specs=[
            pl.BlockSpec((1, 1, H, W, cout), lambda n, d: (n, d, 0, 0, 0)),
            pl.BlockSpec((1, 1, cout), lambda n, d: (n, 0, 0)),
            pl.BlockSpec((1, 1, cout), lambda n, d: (n, 0, 0)),
        ],
        out_shape=[
            jax.ShapeDtypeStruct((N, Dc, H, W, cout), jnp.bfloat16),
            jax.ShapeDtypeStruct((N, 1, cout), jnp.float32),
            jax.ShapeDtypeStruct((N, 1, cout), jnp.float32),
        ],
        scratch_shapes=[
            pltpu.VMEM((3, H + 2, Wp, Cin), jnp.bfloat16),
            pltpu.VMEM((H, Wp, 9 * Cin), jnp.bfloat16),
        ],
        compiler_params=pltpu.CompilerParams(
            dimension_semantics=("parallel", "arbitrary"),
            vmem_limit_bytes=_VMEM_LIMIT),
    )(x5, x5, x5, in_scale, in_shift, wn, bp)

    return y, jnp.sum(s1, axis=0), jnp.sum(s2, axis=0)


def _pack_weights(w):
    cout, cin = w.shape[0], w.shape[1]
    wk = jnp.transpose(w, (2, 3, 1, 4, 0))
    return wk.reshape(9 * cin, 3 * cout).astype(jnp.bfloat16)


def _bn_fold(ysum, ysq, m, gamma, beta, eps=1e-5):
    cout = gamma.shape[0]
    mean = (ysum / m).reshape(1, cout)
    var = (ysq / m).reshape(1, cout) - mean * mean
    scale = gamma.astype(jnp.float32).reshape(1, cout) * lax.rsqrt(var + eps)
    shift = beta.astype(jnp.float32).reshape(1, cout) - mean * scale
    return scale, shift


def kernel(x, w1, b1, g1, bt1, w2, b2, g2, bt2):
    p = _pool_ncdhw_to_ndhwc(x)
    N, D2, H2, W2, _ = p.shape
    M = N * D2 * H2 * W2
    c1, c2 = w1.shape[0], w2.shape[0]

    y1, s1, q1 = _conv3d_bn_stats(p, _pack_weights(w1), b1, c1)
    scale1, shift1 = _bn_fold(s1, q1, M, g1, bt1)

    y2, s2, q2 = _conv3d_bn_stats(y1, _pack_weights(w2), b2, c2,
                                  in_scale=scale1, in_shift=shift1)
    scale2, shift2 = _bn_fold(s2, q2, M, g2, bt2)

    out = jnp.maximum(y2.astype(jnp.float32) * scale2 + shift2, 0.0)
    return jnp.transpose(out, (0, 4, 1, 2, 3))


# speedup vs baseline: 1.6813x; 1.2166x over previous
import functools

import jax
import jax.numpy as jnp
from jax import lax
from jax.experimental import pallas as pl
from jax.experimental.pallas import tpu as pltpu

_VMEM_LIMIT = 64 * 1024 * 1024


def _round_up(x, m):
    return (x + m - 1) // m * m


def _pool_kernel(x_ref, o_ref):
    C = x_ref.shape[1]
    H, W = x_ref.shape[4], x_ref.shape[5]
    DB = o_ref.shape[1]
    for dp in range(DB):
        t = x_ref[0, :, 0, 2 * dp:2 * dp + 2]
        a = jnp.maximum(t[:, 0], t[:, 1]).astype(jnp.bfloat16)
        a = a.reshape(C, H // 2, 2, W)
        a = jnp.maximum(a[:, :, 0], a[:, :, 1])
        b = jnp.transpose(a, (1, 2, 0))
        b = b.reshape(H // 2, W // 2, 2, C)
        o_ref[0, dp] = jnp.maximum(b[:, :, 0], b[:, :, 1])


def _pool_ncdhw_to_ndhwc(x, db=4):
    N, C, D, H, W = x.shape
    D2, H2, W2 = D // 2, H // 2, W // 2
    x6 = x.reshape(N, C, D2 // db, db * 2, H, W)
    return pl.pallas_call(
        _pool_kernel,
        grid=(N, D2 // db),
        in_specs=[pl.BlockSpec((1, C, 1, db * 2, H, W),
                               lambda n, d: (n, 0, d, 0, 0, 0))],
        out_specs=pl.BlockSpec((1, db, H2, W2, C),
                               lambda n, d: (n, d, 0, 0, 0)),
        out_shape=jax.ShapeDtypeStruct((N, D2, H2, W2, C), jnp.bfloat16),
        compiler_params=pltpu.CompilerParams(
            dimension_semantics=("parallel", "parallel"),
            vmem_limit_bytes=_VMEM_LIMIT),
    )(x6)


def _conv_kernel(*refs, apply_norm, db, dc):
    slab_refs = refs[:db + 2]
    sc_ref, sh_ref, w_ref, b_ref, y_ref, sum_ref, sq_ref, p_ref, s_ref = \
        refs[db + 2:]
    dblk = pl.program_id(1)
    H, W, Cout = y_ref.shape[2], y_ref.shape[3], y_ref.shape[4]
    Cin = p_ref.shape[-1]
    Hp, Wp = p_ref.shape[1], p_ref.shape[2]
    M = db * Hp * Wp

    @pl.when(dblk == 0)
    def _():
        p_ref[...] = jnp.zeros_like(p_ref)
        sum_ref[...] = jnp.zeros_like(sum_ref)
        sq_ref[...] = jnp.zeros_like(sq_ref)

    for i, slab_ref in enumerate(slab_refs):
        s = slab_ref[0, 0]
        if apply_norm:
            s = jnp.maximum(s.astype(jnp.float32) * sc_ref[...]
                            + sh_ref[...], 0.0)
        dd = dblk * db + i - 1
        valid = jnp.logical_and(dd >= 0, dd < dc).astype(jnp.float32)
        p_ref[i, 1:H + 1, 1:W + 1, :] = (s * valid).astype(p_ref.dtype)

    for kd in range(3):
        flat = p_ref[kd:kd + db + 1].reshape((db + 1) * Hp * Wp, Cin)
        for kh in range(3):
            j = kd * 3 + kh
            s_ref[:, j * Cin:(j + 1) * Cin] = flat[kh * Wp:kh * Wp + M]

    R = jnp.dot(s_ref[...], w_ref[...], preferred_element_type=jnp.float32)
    V = R.reshape(db, Hp, Wp, 3 * Cout)
    acc = (V[:, 0:H, 0:W, 0:Cout]
           + V[:, 0:H, 1:W + 1, Cout:2 * Cout]
           + V[:, 0:H, 2:W + 2, 2 * Cout:3 * Cout])
    y = acc + b_ref[...]
    y_ref[0] = y.astype(y_ref.dtype)

    sum_ref[...] += jnp.sum(y, axis=(0, 1, 2)).reshape(1, 1, Cout)
    sq_ref[...] += jnp.sum(y * y, axis=(0, 1, 2)).reshape(1, 1, Cout)


def _conv3d_bn_stats(x5, wn, b, cout, in_scale=None, in_shift=None, db=4):
    N, Dc, H, W, Cin = x5.shape
    Hp, Wp = H + 2, _round_up(W + 2, 8)

    apply_norm = in_scale is not None
    if not apply_norm:
        in_scale = jnp.ones((1, Cin), jnp.float32)
        in_shift = jnp.zeros((1, Cin), jnp.float32)
    bp = b.astype(jnp.float32).reshape(1, cout)

    def slab_spec(off):
        return pl.BlockSpec(
            (1, 1, H, W, Cin),
            lambda n, d, o=off: (n, jnp.clip(d * db + o, 0, Dc - 1), 0, 0, 0))

    y, s1, s2 = pl.pallas_call(
        functools.partial(_conv_kernel, apply_norm=apply_norm, db=db, dc=Dc),
        grid=(N, Dc // db),
        in_specs=[slab_spec(o) for o in range(-1, db + 1)] + [
            pl.BlockSpec((1, Cin), lambda n, d: (0, 0)),
            pl.BlockSpec((1, Cin), lambda n, d: (0, 0)),
            pl.BlockSpec((9 * Cin, 3 * cout), lambda n, d: (0, 0)),
            pl.BlockSpec((1, cout), lambda n, d: (0, 0)),
        ],
        out_specs=[
            pl.BlockSpec((1, db, H, W, cout), lambda n, d: (n, d, 0, 0, 0)),
            pl.BlockSpec((1, 1, cout), lambda n, d: (n, 0, 0)),
            pl.BlockSpec((1, 1, cout), lambda n, d: (n, 0, 0)),
        ],
        out_shape=[
            jax.ShapeDtypeStruct((N, Dc, H, W, cout), jnp.bfloat16),
            jax.ShapeDtypeStruct((N, 1, cout), jnp.float32),
            jax.ShapeDtypeStruct((N, 1, cout), jnp.float32),
        ],
        scratch_shapes=[
            pltpu.VMEM((db + 3, Hp, Wp, Cin), jnp.bfloat16),
            pltpu.VMEM((db * Hp * Wp, 9 * Cin), jnp.bfloat16),
        ],
        compiler_params=pltpu.CompilerParams(
            dimension_semantics=("parallel", "arbitrary"),
            vmem_limit_bytes=_VMEM_LIMIT),
    )(*([x5] * (db + 2)), in_scale, in_shift, wn, bp)

    return y, jnp.sum(s1, axis=0), jnp.sum(s2, axis=0)


def _pack_weights(w):
    cout, cin = w.shape[0], w.shape[1]
    wk = jnp.transpose(w, (2, 3, 1, 4, 0))
    return wk.reshape(9 * cin, 3 * cout).astype(jnp.bfloat16)


def _bn_fold(ysum, ysq, m, gamma, beta, eps=1e-5):
    cout = gamma.shape[0]
    mean = (ysum / m).reshape(1, cout)
    var = (ysq / m).reshape(1, cout) - mean * mean
    scale = gamma.astype(jnp.float32).reshape(1, cout) * lax.rsqrt(var + eps)
    shift = beta.astype(jnp.float32).reshape(1, cout) - mean * scale
    return scale, shift


def kernel(x, w1, b1, g1, bt1, w2, b2, g2, bt2):
    p = _pool_ncdhw_to_ndhwc(x)
    N, D2, H2, W2, _ = p.shape
    M = N * D2 * H2 * W2
    c1, c2 = w1.shape[0], w2.shape[0]

    y1, s1, q1 = _conv3d_bn_stats(p, _pack_weights(w1), b1, c1)
    scale1, shift1 = _bn_fold(s1, q1, M, g1, bt1)

    y2, s2, q2 = _conv3d_bn_stats(y1, _pack_weights(w2), b2, c2,
                                  in_scale=scale1, in_shift=shift1)
    scale2, shift2 = _bn_fold(s2, q2, M, g2, bt2)

    out = jnp.maximum(y2.astype(jnp.float32) * scale2 + shift2, 0.0)
    return jnp.transpose(out, (0, 4, 1, 2, 3))
